# baseline (speedup 1.0000x reference)
import jax
import jax.numpy as jnp
from jax.experimental import pallas as pl
from jax.experimental.pallas import tpu as pltpu

_B, _A, _NC, _DFL = 32, 8400, 80, 16
_NCH = 4 * (_DFL + 1)


def _loss_kernel(pd_ref, ts_ref, tb_ref, ap_ref, pb_ref, m_ref,
                 tss_ref, box_ref, dfl_ref, accb_ref, accd_ref):
    b = pl.program_id(0)
    f32 = jnp.float32

    @pl.when(b == 0)
    def _init():
        accb_ref[...] = jnp.zeros_like(accb_ref)
        accd_ref[...] = jnp.zeros_like(accd_ref)

    P = pd_ref[0]
    TS = ts_ref[0]
    TB = tb_ref[0]
    AP = ap_ref[...]
    PB = pb_ref[0]
    mrow = m_ref[0]

    lane_contract = (((1,), (1,)), ((), ()))

    i4 = jax.lax.broadcasted_iota(jnp.int32, (4, 4), 0)
    j4 = jax.lax.broadcasted_iota(jnp.int32, (4, 4), 1)
    I4 = jnp.where(i4 == j4, 1.0, 0.0).astype(f32)
    pbt = jax.lax.dot_general(I4, PB, lane_contract,
                              preferred_element_type=f32)
    tbt = jax.lax.dot_general(I4, TB, lane_contract,
                              preferred_element_type=f32)

    w = jax.lax.dot_general(jnp.ones((1, _NC), f32), TS, lane_contract,
                            preferred_element_type=f32)
    w = w * mrow

    ix = jnp.minimum(pbt[2:3], tbt[2:3]) - jnp.maximum(pbt[0:1], tbt[0:1])
    iy = jnp.minimum(pbt[3:4], tbt[3:4]) - jnp.maximum(pbt[1:2], tbt[1:2])
    inter = jnp.maximum(ix, 0.0) * jnp.maximum(iy, 0.0)
    area1 = (pbt[2:3] - pbt[0:1]) * (pbt[3:4] - pbt[1:2])
    area2 = (tbt[2:3] - tbt[0:1]) * (tbt[3:4] - tbt[1:2])
    iou = inter / (area1 + area2 - inter + 1e-7)
    accb_ref[...] += (1.0 - iou) * w

    lane68 = jax.lax.broadcasted_iota(jnp.int32, (1, _NCH), 1)
    g1 = lane68 // (_DFL + 1)
    sgn = jnp.where(g1 >= 2, 1.0, -1.0).astype(f32)
    r4 = jax.lax.broadcasted_iota(jnp.int32, (4, _NCH), 0)
    g4 = jax.lax.broadcasted_iota(jnp.int32, (4, _NCH), 1) // (_DFL + 1)
    G4 = jnp.where(r4 == g4, 1.0, 0.0).astype(f32)
    M1 = G4 * sgn
    r2 = jax.lax.broadcasted_iota(jnp.int32, (2, _NCH), 0)
    g2 = jax.lax.broadcasted_iota(jnp.int32, (2, _NCH), 1) // (_DFL + 1)
    M2 = jnp.where(r2 == (g2 % 2), 1.0, 0.0).astype(f32) * (-sgn)

    row_contract = (((1,), (0,)), ((), ()))
    t68 = (jax.lax.dot_general(TB, M1, row_contract, preferred_element_type=f32)
           + jax.lax.dot_general(AP, M2, row_contract, preferred_element_type=f32))
    t68 = jnp.clip(t68, 0.0, _DFL - 0.01)

    l17 = (lane68 % (_DFL + 1)).astype(f32)
    coef = jnp.maximum(1.0 - jnp.abs(t68 - l17), 0.0)

    E = jnp.exp(jnp.clip(P, -85.0, 85.0))
    S = jax.lax.dot_general(G4, E, lane_contract,
                            preferred_element_type=f32)
    lse = jnp.sum(jnp.log(S), axis=0, keepdims=True)

    sel = jax.lax.dot_general(jnp.ones((1, _NCH), f32), P * coef,
                              lane_contract,
                              preferred_element_type=f32)
    accd_ref[...] += (lse - sel) * 0.25 * w

    @pl.when(b == _B - 1)
    def _finalize():
        inv = 1.0 / tss_ref[0, 0]
        box_ref[...] = jnp.reshape(jnp.sum(accb_ref[...]) * inv, (1, 1))
        dfl_ref[...] = jnp.reshape(jnp.sum(accd_ref[...]) * inv, (1, 1))


def kernel(pred_dist, pred_bboxes, anchor_points, target_bboxes,
           target_scores, target_scores_sum, fg_mask):
    f32 = jnp.float32
    mask = fg_mask.astype(f32).reshape(_B, 1, _A)
    tss = target_scores_sum.reshape(1, 1)
    out = pl.pallas_call(
        _loss_kernel,
        grid=(_B,),
        in_specs=[
            pl.BlockSpec((1, _A, _NCH), lambda b: (b, 0, 0)),
            pl.BlockSpec((1, _A, _NC), lambda b: (b, 0, 0)),
            pl.BlockSpec((1, _A, 4), lambda b: (b, 0, 0)),
            pl.BlockSpec((_A, 2), lambda b: (0, 0)),
            pl.BlockSpec((1, _A, 4), lambda b: (b, 0, 0)),
            pl.BlockSpec((1, 1, _A), lambda b: (b, 0, 0)),
            pl.BlockSpec((1, 1), lambda b: (0, 0)),
        ],
        out_specs=[
            pl.BlockSpec((1, 1), lambda b: (0, 0)),
            pl.BlockSpec((1, 1), lambda b: (0, 0)),
        ],
        out_shape=[jax.ShapeDtypeStruct((1, 1), f32),
                   jax.ShapeDtypeStruct((1, 1), f32)],
        scratch_shapes=[pltpu.VMEM((1, _A), f32),
                        pltpu.VMEM((1, _A), f32)],
    )(pred_dist, target_scores, target_bboxes, anchor_points, pred_bboxes,
      mask, tss)
    return (out[0][0, 0], out[1][0, 0])


# speedup vs baseline: 2.4448x; 2.4448x over previous
import jax
import jax.numpy as jnp
from jax.experimental import pallas as pl
from jax.experimental.pallas import tpu as pltpu

_B, _A, _NC, _DFL = 32, 8400, 80, 16
_NCH = 4 * (_DFL + 1)


def _loss_kernel(pd_ref, ts_ref, bx_ref, ap_ref, m_ref, tss_ref,
                 box_ref, dfl_ref, accb_ref, accd_ref):
    b = pl.program_id(0)
    f32 = jnp.float32

    @pl.when(b == 0)
    def _init():
        accb_ref[...] = jnp.zeros_like(accb_ref)
        accd_ref[...] = jnp.zeros_like(accd_ref)

    P = pd_ref[0]
    TS = ts_ref[0]
    bx = bx_ref[0]
    apx = ap_ref[0:1]
    apy = ap_ref[1:2]
    mrow = m_ref[0]

    w = jax.lax.dot_general(jnp.ones((1, _NC), f32), TS, (((1,), (0,)), ((), ())),
                            preferred_element_type=f32)
    w = w * mrow

    ix = jnp.minimum(bx[2:3], bx[6:7]) - jnp.maximum(bx[0:1], bx[4:5])
    iy = jnp.minimum(bx[3:4], bx[7:8]) - jnp.maximum(bx[1:2], bx[5:6])
    inter = jnp.maximum(ix, 0.0) * jnp.maximum(iy, 0.0)
    area1 = (bx[2:3] - bx[0:1]) * (bx[3:4] - bx[1:2])
    area2 = (bx[6:7] - bx[4:5]) * (bx[7:8] - bx[5:6])
    iou = inter / (area1 + area2 - inter + 1e-7)
    accb_ref[...] += (1.0 - iou) * w

    t4 = jnp.concatenate([apx - bx[4:5], apy - bx[5:6],
                          bx[6:7] - apx, bx[7:8] - apy], axis=0)
    t4 = jnp.clip(t4, 0.0, _DFL - 0.01)
    lane68 = jax.lax.broadcasted_iota(jnp.int32, (1, _NCH), 1)
    r4 = jax.lax.broadcasted_iota(jnp.int32, (4, _NCH), 0)
    g4 = jax.lax.broadcasted_iota(jnp.int32, (4, _NCH), 1) // (_DFL + 1)
    G4 = jnp.where(r4 == g4, 1.0, 0.0).astype(f32)
    t68 = jax.lax.dot_general(t4, G4, (((0,), (0,)), ((), ())),
                              preferred_element_type=f32)

    l17 = (lane68 % (_DFL + 1)).astype(f32)
    coef = jnp.maximum(1.0 - jnp.abs(t68 - l17), 0.0)

    lane_contract = (((1,), (1,)), ((), ()))
    E = jnp.exp(jnp.clip(P, -85.0, 85.0))
    S = jax.lax.dot_general(G4, E, lane_contract,
                            preferred_element_type=f32)
    lse = jnp.sum(jnp.log(S), axis=0, keepdims=True)

    sel = jax.lax.dot_general(jnp.ones((1, _NCH), f32), P * coef,
                              lane_contract,
                              preferred_element_type=f32)
    accd_ref[...] += (lse - sel) * 0.25 * w

    @pl.when(b == _B - 1)
    def _finalize():
        inv = 1.0 / tss_ref[0, 0]
        box_ref[...] = jnp.reshape(jnp.sum(accb_ref[...]) * inv, (1, 1))
        dfl_ref[...] = jnp.reshape(jnp.sum(accd_ref[...]) * inv, (1, 1))


def kernel(pred_dist, pred_bboxes, anchor_points, target_bboxes,
           target_scores, target_scores_sum, fg_mask):
    f32 = jnp.float32
    tst = jnp.transpose(target_scores, (0, 2, 1))
    bxp = jnp.concatenate([jnp.swapaxes(pred_bboxes, 1, 2),
                           jnp.swapaxes(target_bboxes, 1, 2)], axis=1)
    apt = jnp.transpose(anchor_points)
    mask = fg_mask.astype(f32).reshape(_B, 1, _A)
    tss = target_scores_sum.reshape(1, 1)
    out = pl.pallas_call(
        _loss_kernel,
        grid=(_B,),
        in_specs=[
            pl.BlockSpec((1, _A, _NCH), lambda b: (b, 0, 0)),
            pl.BlockSpec((1, _NC, _A), lambda b: (b, 0, 0)),
            pl.BlockSpec((1, 8, _A), lambda b: (b, 0, 0)),
            pl.BlockSpec((2, _A), lambda b: (0, 0)),
            pl.BlockSpec((1, 1, _A), lambda b: (b, 0, 0)),
            pl.BlockSpec((1, 1), lambda b: (0, 0)),
        ],
        out_specs=[
            pl.BlockSpec((1, 1), lambda b: (0, 0)),
            pl.BlockSpec((1, 1), lambda b: (0, 0)),
        ],
        out_shape=[jax.ShapeDtypeStruct((1, 1), f32),
                   jax.ShapeDtypeStruct((1, 1), f32)],
        scratch_shapes=[pltpu.VMEM((1, _A), f32),
                        pltpu.VMEM((1, _A), f32)],
    )(pred_dist, tst, bxp, apt, mask, tss)
    return (out[0][0, 0], out[1][0, 0])


# speedup vs baseline: 5.7526x; 2.3530x over previous
import jax
import jax.numpy as jnp
from jax.experimental import pallas as pl
from jax.experimental.pallas import tpu as pltpu

_B, _A, _NC, _DFL = 32, 8400, 80, 16
_NCH = 4 * (_DFL + 1)
_BB = 8


def _dfl_kernel(pd_ref, bx_ref, ap_ref, out_ref, acc_ref, t4_ref):
    g = pl.program_id(1)

    @pl.when(g == 0)
    def _prep():
        apx = ap_ref[0:1]
        apy = ap_ref[1:2]
        hi = _DFL - 0.01
        t4_ref[0] = jnp.clip(apx - bx_ref[:, 4, :], 0.0, hi)
        t4_ref[1] = jnp.clip(apy - bx_ref[:, 5, :], 0.0, hi)
        t4_ref[2] = jnp.clip(bx_ref[:, 6, :] - apx, 0.0, hi)
        t4_ref[3] = jnp.clip(bx_ref[:, 7, :] - apy, 0.0, hi)
        acc_ref[...] = jnp.zeros_like(acc_ref)

    P = pd_ref[...]
    E = jnp.exp(jnp.clip(P, -85.0, 85.0))
    S = jnp.sum(E, axis=0)
    t_g = t4_ref[pl.ds(g, 1)][0]
    c17 = jax.lax.broadcasted_iota(jnp.int32, (_DFL + 1, 1, 1), 0)
    coef = jnp.maximum(1.0 - jnp.abs(t_g[None] - c17.astype(jnp.float32)), 0.0)
    sel = jnp.sum(P * coef, axis=0)
    acc_ref[...] += jnp.log(S) - sel

    @pl.when(g == 3)
    def _flush():
        out_ref[:, 0, :] = acc_ref[...]


def _loss_kernel(ts_ref, bx_ref, dflr_ref, m_ref, tss_ref,
                 box_ref, dfl_ref, accb_ref, accd_ref):
    b = pl.program_id(0)
    f32 = jnp.float32

    @pl.when(b == 0)
    def _init():
        accb_ref[...] = jnp.zeros_like(accb_ref)
        accd_ref[...] = jnp.zeros_like(accd_ref)

    TS = ts_ref[0]
    bx = bx_ref[0]
    mrow = m_ref[0]

    w = jax.lax.dot_general(jnp.ones((1, _NC), f32), TS, (((1,), (0,)), ((), ())),
                            preferred_element_type=f32)
    w = w * mrow

    ix = jnp.minimum(bx[2:3], bx[6:7]) - jnp.maximum(bx[0:1], bx[4:5])
    iy = jnp.minimum(bx[3:4], bx[7:8]) - jnp.maximum(bx[1:2], bx[5:6])
    inter = jnp.maximum(ix, 0.0) * jnp.maximum(iy, 0.0)
    area1 = (bx[2:3] - bx[0:1]) * (bx[3:4] - bx[1:2])
    area2 = (bx[6:7] - bx[4:5]) * (bx[7:8] - bx[5:6])
    iou = inter / (area1 + area2 - inter + 1e-7)
    accb_ref[...] += (1.0 - iou) * w

    accd_ref[...] += dflr_ref[0] * 0.25 * w

    @pl.when(b == _B - 1)
    def _finalize():
        inv = 1.0 / tss_ref[0, 0]
        box_ref[...] = jnp.reshape(jnp.sum(accb_ref[...]) * inv, (1, 1))
        dfl_ref[...] = jnp.reshape(jnp.sum(accd_ref[...]) * inv, (1, 1))


def kernel(pred_dist, pred_bboxes, anchor_points, target_bboxes,
           target_scores, target_scores_sum, fg_mask):
    f32 = jnp.float32
    pdt = jnp.transpose(pred_dist, (2, 0, 1))
    tst = jnp.transpose(target_scores, (0, 2, 1))
    bxp = jnp.concatenate([jnp.swapaxes(pred_bboxes, 1, 2),
                           jnp.swapaxes(target_bboxes, 1, 2)], axis=1)
    apt = jnp.transpose(anchor_points)
    mask = fg_mask.astype(f32).reshape(_B, 1, _A)
    tss = target_scores_sum.reshape(1, 1)

    dfl_raw = pl.pallas_call(
        _dfl_kernel,
        grid=(_B // _BB, 4),
        in_specs=[
            pl.BlockSpec((_DFL + 1, _BB, _A), lambda bb, g: (g, bb, 0)),
            pl.BlockSpec((_BB, 8, _A), lambda bb, g: (bb, 0, 0)),
            pl.BlockSpec((2, _A), lambda bb, g: (0, 0)),
        ],
        out_specs=pl.BlockSpec((_BB, 1, _A), lambda bb, g: (bb, 0, 0)),
        out_shape=jax.ShapeDtypeStruct((_B, 1, _A), f32),
        scratch_shapes=[pltpu.VMEM((_BB, _A), f32),
                        pltpu.VMEM((4, _BB, _A), f32)],
    )(pdt, bxp, apt)

    out = pl.pallas_call(
        _loss_kernel,
        grid=(_B,),
        in_specs=[
            pl.BlockSpec((1, _NC, _A), lambda b: (b, 0, 0)),
            pl.BlockSpec((1, 8, _A), lambda b: (b, 0, 0)),
            pl.BlockSpec((1, 1, _A), lambda b: (b, 0, 0)),
            pl.BlockSpec((1, 1, _A), lambda b: (b, 0, 0)),
            pl.BlockSpec((1, 1), lambda b: (0, 0)),
        ],
        out_specs=[
            pl.BlockSpec((1, 1), lambda b: (0, 0)),
            pl.BlockSpec((1, 1), lambda b: (0, 0)),
        ],
        out_shape=[jax.ShapeDtypeStruct((1, 1), f32),
                   jax.ShapeDtypeStruct((1, 1), f32)],
        scratch_shapes=[pltpu.VMEM((1, _A), f32),
                        pltpu.VMEM((1, _A), f32)],
    )(tst, bxp, dfl_raw, mask, tss)
    return (out[0][0, 0], out[1][0, 0])


# speedup vs baseline: 5.7987x; 1.0080x over previous
import jax
import jax.numpy as jnp
from jax.experimental import pallas as pl
from jax.experimental.pallas import tpu as pltpu

_B, _A, _NC, _DFL = 32, 8400, 80, 16
_NCH = 4 * (_DFL + 1)
_BB = 8
_NBB = _B // _BB
_TSC = 16
_NI = 4 + _NC // _TSC


def _loss_kernel(pd_ref, ts_ref, bx_ref, ap_ref, m_ref, tss_ref,
                 box_ref, dfl_ref, accd_ref, t4_ref, w_ref):
    bb = pl.program_id(0)
    i = pl.program_id(1)
    f32 = jnp.float32

    @pl.when(jnp.logical_and(bb == 0, i == 0))
    def _init_out():
        box_ref[...] = jnp.zeros_like(box_ref)
        dfl_ref[...] = jnp.zeros_like(dfl_ref)

    @pl.when(i == 0)
    def _prep():
        apx = ap_ref[0:1]
        apy = ap_ref[1:2]
        hi = _DFL - 0.01
        t4_ref[0] = jnp.clip(apx - bx_ref[:, 4, :], 0.0, hi)
        t4_ref[1] = jnp.clip(apy - bx_ref[:, 5, :], 0.0, hi)
        t4_ref[2] = jnp.clip(bx_ref[:, 6, :] - apx, 0.0, hi)
        t4_ref[3] = jnp.clip(bx_ref[:, 7, :] - apy, 0.0, hi)
        accd_ref[...] = jnp.zeros_like(accd_ref)

    @pl.when(i < 4)
    def _dfl_group():
        P = pd_ref[...]
        E = jnp.exp(jnp.clip(P, -85.0, 85.0))
        S = jnp.sum(E, axis=0)
        t_g = t4_ref[pl.ds(i, 1)][0]
        c17 = jax.lax.broadcasted_iota(jnp.int32, (_DFL + 1, 1, 1), 0)
        coef = jnp.maximum(1.0 - jnp.abs(t_g[None] - c17.astype(f32)), 0.0)
        sel = jnp.sum(P * coef, axis=0)
        accd_ref[...] += jnp.log(S) - sel

    @pl.when(i == 4)
    def _w_init():
        w_ref[...] = jnp.sum(ts_ref[...], axis=1)

    @pl.when(i > 4)
    def _w_acc():
        w_ref[...] += jnp.sum(ts_ref[...], axis=1)

    @pl.when(i == _NI - 1)
    def _combine():
        bx = bx_ref[...]
        ix = (jnp.minimum(bx[:, 2, :], bx[:, 6, :])
              - jnp.maximum(bx[:, 0, :], bx[:, 4, :]))
        iy = (jnp.minimum(bx[:, 3, :], bx[:, 7, :])
              - jnp.maximum(bx[:, 1, :], bx[:, 5, :]))
        inter = jnp.maximum(ix, 0.0) * jnp.maximum(iy, 0.0)
        area1 = ((bx[:, 2, :] - bx[:, 0, :]) * (bx[:, 3, :] - bx[:, 1, :]))
        area2 = ((bx[:, 6, :] - bx[:, 4, :]) * (bx[:, 7, :] - bx[:, 5, :]))
        iou = inter / (area1 + area2 - inter + 1e-7)
        w = w_ref[...] * m_ref[:, 0, :]
        box_part = jnp.sum((1.0 - iou) * w)
        dfl_part = jnp.sum(accd_ref[...] * 0.25 * w)
        box_ref[...] += jnp.reshape(box_part, (1, 1))
        dfl_ref[...] += jnp.reshape(dfl_part, (1, 1))

        @pl.when(bb == _NBB - 1)
        def _finalize():
            inv = 1.0 / tss_ref[0, 0]
            box_ref[...] *= inv
            dfl_ref[...] *= inv


def kernel(pred_dist, pred_bboxes, anchor_points, target_bboxes,
           target_scores, target_scores_sum, fg_mask):
    f32 = jnp.float32
    pdt = jnp.transpose(pred_dist, (2, 0, 1))
    tst = jnp.transpose(target_scores, (0, 2, 1))
    bxp = jnp.concatenate([jnp.swapaxes(pred_bboxes, 1, 2),
                           jnp.swapaxes(target_bboxes, 1, 2)], axis=1)
    apt = jnp.transpose(anchor_points)
    mask = fg_mask.astype(f32).reshape(_B, 1, _A)
    tss = target_scores_sum.reshape(1, 1)

    out = pl.pallas_call(
        _loss_kernel,
        grid=(_NBB, _NI),
        in_specs=[
            pl.BlockSpec((_DFL + 1, _BB, _A),
                         lambda bb, i: (jnp.minimum(i, 3), bb, 0)),
            pl.BlockSpec((_BB, _TSC, _A),
                         lambda bb, i: (bb, jnp.maximum(i - 4, 0), 0)),
            pl.BlockSpec((_BB, 8, _A), lambda bb, i: (bb, 0, 0)),
            pl.BlockSpec((2, _A), lambda bb, i: (0, 0)),
            pl.BlockSpec((_BB, 1, _A), lambda bb, i: (bb, 0, 0)),
            pl.BlockSpec((1, 1), lambda bb, i: (0, 0)),
        ],
        out_specs=[
            pl.BlockSpec((1, 1), lambda bb, i: (0, 0)),
            pl.BlockSpec((1, 1), lambda bb, i: (0, 0)),
        ],
        out_shape=[jax.ShapeDtypeStruct((1, 1), f32),
                   jax.ShapeDtypeStruct((1, 1), f32)],
        scratch_shapes=[pltpu.VMEM((_BB, _A), f32),
                        pltpu.VMEM((4, _BB, _A), f32),
                        pltpu.VMEM((_BB, _A), f32)],
    )(pdt, tst, bxp, apt, mask, tss)
    return (out[0][0, 0], out[1][0, 0])
